# baseline (speedup 1.0000x reference)
import functools

import jax
import jax.numpy as jnp
from jax import lax
from jax.experimental import pallas as pl
from jax.experimental.pallas import tpu as pltpu
from jax.experimental.pallas import tpu_sc as plsc

B = 4096
F = 100
D = 128
TOT = B * F
NC, NS, L = 2, 16, 16
NW = NC * NS
PER_W = TOT // NW
CHUNK = 128
NBUF = 5
LOOKAHEAD = 3
NCHUNK = PER_W // CHUNK
NVEC = PER_W // L
BLK_PER_F = B // L


def _sc_gather(features_fmajor, table, off_rep):
  mesh = plsc.VectorSubcoreMesh(core_axis_name="c", subcore_axis_name="s")

  @functools.partial(
      pl.kernel,
      out_type=jax.ShapeDtypeStruct((TOT, D), jnp.float32),
      mesh=mesh,
      scratch_types=[
          pltpu.VMEM((PER_W,), jnp.int32),
          pltpu.VMEM((F * L,), jnp.int32),
          [pltpu.VMEM((CHUNK, D), jnp.float32) for _ in range(NBUF)],
          [pltpu.SemaphoreType.DMA for _ in range(NBUF)],
          [pltpu.SemaphoreType.DMA for _ in range(NBUF)],
      ],
  )
  def k(feat_hbm, table_hbm, off_hbm, out_hbm, idx_v, off_v, bufs, gsems, wsems):
    wid = lax.axis_index("s") * NC + lax.axis_index("c")
    base = wid * PER_W

    pltpu.sync_copy(feat_hbm.at[pl.ds(base, PER_W)], idx_v)
    pltpu.sync_copy(off_hbm, off_v)

    blk0 = wid * NVEC
    BLK_PER_C = CHUNK // L

    def add_chunk(c):
      for t in range(BLK_PER_C):
        j = c * BLK_PER_C + t
        fld = (blk0 + j) // BLK_PER_F
        s = pl.ds(j * L, L)
        idx_v[s] = idx_v[s] + off_v[pl.ds(fld * L, L)]

    def gather_desc(c, b):
      return pltpu.make_async_copy(
          table_hbm.at[idx_v.at[pl.ds(c * CHUNK, CHUNK)]], bufs[b], gsems[b])

    def wb_desc(c, b):
      return pltpu.make_async_copy(
          bufs[b], out_hbm.at[pl.ds(base + c * CHUNK, CHUNK)], wsems[b])

    for c in range(LOOKAHEAD):
      add_chunk(c)
      gather_desc(c, c % NBUF).start()

    def group_body(g, _):
      for b in range(NBUF):
        c = g * NBUF + b
        ca = c + LOOKAHEAD
        ba = (b + LOOKAHEAD) % NBUF
        cd = ca - NBUF

        @pl.when((ca < NCHUNK) & (cd >= 0))
        def _():
          wb_desc(cd, ba).wait()

        @pl.when(ca < NCHUNK)
        def _():
          add_chunk(ca)
          gather_desc(ca, ba).start()

        gather_desc(c, b).wait()
        wb_desc(c, b).start()
      return 0

    lax.fori_loop(0, NCHUNK // NBUF, group_body, 0)

    for cc in range(NCHUNK - NBUF, NCHUNK):
      wb_desc(cc, cc % NBUF).wait()

  return k(features_fmajor, table, off_rep)


def kernel(features, table, feature_offsets):
  feats_fmajor = jnp.transpose(features).reshape(TOT)
  off_rep = jnp.repeat(feature_offsets, L)
  out = _sc_gather(feats_fmajor, table, off_rep)
  return out.reshape(F, B, D).transpose(1, 0, 2)
